# baseline (speedup 1.0000x reference)
import functools

import jax
import jax.numpy as jnp
from jax import lax
from jax.experimental import pallas as pl
from jax.experimental.pallas import tpu as pltpu
from jax.experimental.pallas import tpu_sc as plsc

BATCH = 16384
D = 128
BINS = 32
NENV = 64
ROWS = 1024
NBLK = BATCH // ROWS

NW = 32
NBUCKET = 131072
BPW = NBUCKET // NW
KWAY = 4
L = 16
NCHUNK = BATCH // L


def _stats_kernel(f_ref, s1_ref, s2_ref):
    b = pl.program_id(0)
    x = f_ref[...]
    s1 = jnp.sum(x, axis=0)[None, :]
    s2 = jnp.sum(x * x, axis=0)[None, :]

    @pl.when(b == 0)
    def _():
        s1_ref[...] = s1
        s2_ref[...] = s2

    @pl.when(b > 0)
    def _():
        s1_ref[...] += s1
        s2_ref[...] += s2


def _hash_kernel(f_ref, env_ref, mean_ref, inv_ref, rp_ref, e_ref, b_ref,
                 t_ref):
    x = (f_ref[...] - mean_ref[...]) * inv_ref[...]
    p = jnp.dot(x, rp_ref[...], preferred_element_type=jnp.float32)
    sign = jnp.where(p > 0, jnp.float32(1), jnp.float32(-1))
    ks = lax.broadcasted_iota(jnp.int32, (1, NENV), 1)
    env = env_ref[...]
    onehot = jnp.where(env == ks, jnp.float32(8), jnp.float32(0))
    pad = jnp.zeros((ROWS, D - BINS - NENV), jnp.float32)
    e = jnp.concatenate([sign, onehot, pad], axis=1)
    e_ref[...] = e.astype(jnp.bfloat16)

    powers = jnp.left_shift(
        jnp.int32(1), lax.broadcasted_iota(jnp.int32, (1, BINS), 1))
    bits = jnp.where(p > 0, powers, jnp.int32(0))
    h = jnp.sum(bits, axis=1, keepdims=True, dtype=jnp.int32)
    b_ref[...] = jnp.bitwise_and(h, jnp.int32(NBUCKET - 1))
    hi = jnp.right_shift(h.astype(jnp.uint32), jnp.uint32(17)).astype(jnp.int32)
    t_ref[...] = jnp.bitwise_or(hi, jnp.left_shift(env, jnp.int32(15)))


def _count_kernel(ei_ref, ej_ref, out_ref):
    i = pl.program_id(0)
    j = pl.program_id(1)
    nj = pl.num_programs(1)

    @pl.when(j == 0)
    def _():
        out_ref[...] = jnp.ones((ROWS, 1), jnp.float32)

    @pl.when(j < i)
    def _():
        s = lax.dot_general(
            ei_ref[...], ej_ref[...], (((1,), (1,)), ((), ())),
            preferred_element_type=jnp.float32)
        out_ref[...] += jnp.sum((s > 95.0).astype(jnp.float32), axis=1,
                                keepdims=True)

    @pl.when(j == i)
    def _():
        s = lax.dot_general(
            ei_ref[...], ej_ref[...], (((1,), (1,)), ((), ())),
            preferred_element_type=jnp.float32)
        ii = lax.broadcasted_iota(jnp.int32, (ROWS, 1), 0)
        jj = lax.broadcasted_iota(jnp.int32, (1, ROWS), 1)
        eq = (s > 95.0) & (jj < ii)
        out_ref[...] += jnp.sum(eq.astype(jnp.float32), axis=1, keepdims=True)

    @pl.when(j == nj - 1)
    def _():
        out_ref[...] = 1.0 / jnp.sqrt(out_ref[...])


def _tc_count(enc):
    return pl.pallas_call(
        _count_kernel,
        grid=(NBLK, NBLK),
        in_specs=[
            pl.BlockSpec((ROWS, D), lambda i, j: (i, j * 0)),
            pl.BlockSpec((ROWS, D), lambda i, j: (j, i * 0)),
        ],
        out_specs=pl.BlockSpec((ROWS, 1), lambda i, j: (i, j * 0)),
        out_shape=jax.ShapeDtypeStruct((BATCH, 1), jnp.float32),
    )(enc, enc)


def _rsqrt_newton(c):
    ih = plsc.bitcast(c, jnp.int32)
    y = plsc.bitcast(jnp.int32(0x5F3759DF) - jnp.right_shift(ih, jnp.int32(1)),
                     jnp.float32)
    for _ in range(3):
        y = y * (1.5 - 0.5 * c * y * y)
    return y


def _sc_body(b_hbm, t_hbm, idx_hbm, out_hbm, flag_hbm,
             bkt_v, tag_v, rew_v, idx_v, ctab, ttab, stage_v, acc_sh):
    info = plsc.get_sparse_core_info()
    nc = info.num_cores
    core = lax.axis_index("c")
    sid = lax.axis_index("s")
    wid = sid * nc + core

    pltpu.sync_copy(b_hbm, bkt_v)
    pltpu.sync_copy(t_hbm, tag_v)
    pltpu.sync_copy(idx_hbm, idx_v)

    zeros16i = jnp.zeros((L,), jnp.int32)

    def zero_body(z, carry):
        ctab[pl.ds(z * L, L)] = zeros16i
        return carry

    lax.fori_loop(jnp.int32(0), jnp.int32(KWAY * BPW // L), zero_body,
                  jnp.int32(0))

    iota16 = lax.broadcasted_iota(jnp.int32, (L,), 0)

    def chunk_body(c, flag):
        base = c * L
        b = bkt_v[pl.ds(base, L)]
        t = tag_v[pl.ds(base, L)]
        lb = jnp.bitwise_and(b, jnp.int32(BPW - 1))
        mine = jnp.right_shift(b, jnp.int32(12)) == wid

        gcs = []
        gts = []
        for k in range(KWAY):
            gidx = lb + jnp.int32(k * BPW)
            gcs.append(plsc.load_gather(ctab, [gidx]))
            gts.append(plsc.load_gather(ttab, [gidx]))
        occ = [g > 0 for g in gcs]
        match = [occ[k] & (gts[k] == t) for k in range(KWAY)]
        gathered = zeros16i
        found = iota16 < 0
        way = jnp.full((L,), KWAY, jnp.int32)
        for k in range(KWAY - 1, -1, -1):
            gathered = jnp.where(match[k], gcs[k], gathered)
            found = found | match[k]
            way = jnp.where(match[k], jnp.int32(k), way)

        pcv = plsc.all_reduce_population_count(mine)
        pc = jnp.max(pcv)

        def pairwise():
            rank = zeros16i
            after = zeros16i
            dbefore = zeros16i
            for m in range(L):
                sel = jnp.full((L,), m, jnp.int32) + base
                bm = plsc.load_gather(bkt_v, [sel])
                tm = plsc.load_gather(tag_v, [sel])
                beq = b == bm
                eq = beq & (t == tm)
                dif = beq & (t != tm)
                rank = rank + ((iota16 > m) & eq).astype(jnp.int32)
                after = after + ((iota16 < m) & eq).astype(jnp.int32)
                dbefore = dbefore + ((iota16 > m) & dif).astype(jnp.int32)
            return rank, after, dbefore

        def trivial():
            return zeros16i, zeros16i, zeros16i

        rank, after, dbefore = lax.cond(pc > 1, pairwise, trivial)

        is_last = after == 0
        cumempty = zeros16i
        claim = jnp.full((L,), KWAY, jnp.int32)
        for k in range(KWAY):
            cumempty = cumempty + (~occ[k]).astype(jnp.int32)
            take = (~occ[k]) & (cumempty == dbefore + 1) & (claim == KWAY)
            claim = jnp.where(take, jnp.int32(k), claim)
        slot = jnp.where(found, way, claim)
        overflow = mine & (slot == KWAY)
        flag = flag | overflow.astype(jnp.int32)

        newc = gathered + rank + 1
        upd = mine & is_last & (slot < KWAY)
        for k in range(KWAY):
            mk = upd & (slot == k)
            sidx = lb + jnp.int32(k * BPW)
            plsc.store_scatter(ctab, [sidx], newc, mask=mk)
            plsc.store_scatter(ttab, [sidx], t, mask=mk)

        rw = _rsqrt_newton(newc.astype(jnp.float32))
        rew_v[pl.ds(base, L)] = jnp.where(mine, rw, jnp.float32(0))
        return flag

    flag = lax.fori_loop(jnp.int32(0), jnp.int32(NCHUNK), chunk_body,
                         zeros16i)

    stage_v[...] = flag
    pltpu.sync_copy(stage_v, flag_hbm.at[wid])

    plsc.subcore_barrier()

    @pl.when(sid == 0)
    def _():
        for j in range(BATCH // 128):
            pltpu.sync_copy(rew_v.at[pl.ds(jnp.int32(j * 128), 128)],
                            acc_sh.at[idx_v.at[jnp.int32(j)]])

    plsc.subcore_barrier()

    @pl.when(sid != 0)
    def _():
        for j in range(BATCH // 128):
            pltpu.sync_copy(rew_v.at[pl.ds(jnp.int32(j * 128), 128)],
                            acc_sh.at[idx_v.at[jnp.int32(j)]], add=True)

    plsc.subcore_barrier()

    @pl.when(sid == 0)
    def _():
        pltpu.sync_copy(acc_sh, out_hbm.at[core])


def kernel(features, env_indices, random_projection):
    features = features.astype(jnp.float32)
    s1, s2 = pl.pallas_call(
        _stats_kernel,
        grid=(NBLK,),
        in_specs=[pl.BlockSpec((ROWS, D), lambda b: (b, b * 0))],
        out_specs=[
            pl.BlockSpec((1, D), lambda b: (b * 0, b * 0)),
            pl.BlockSpec((1, D), lambda b: (b * 0, b * 0)),
        ],
        out_shape=[
            jax.ShapeDtypeStruct((1, D), jnp.float32),
            jax.ShapeDtypeStruct((1, D), jnp.float32),
        ],
    )(features)

    eps_count = jnp.float32(1e-4)
    bc = jnp.float32(BATCH)
    batch_mean = s1 / bc
    batch_var = (s2 - s1 * s1 / bc) / (bc - 1.0)
    tot = eps_count + bc
    new_mean = batch_mean * (bc / tot)
    m2 = eps_count + batch_var * bc + batch_mean**2 * eps_count * bc / tot
    new_var = m2 / tot
    inv_std = 1.0 / jnp.sqrt(new_var + 1e-8)

    env_col = env_indices.astype(jnp.int32)[:, None]
    enc, bkt, tag = pl.pallas_call(
        _hash_kernel,
        grid=(NBLK,),
        in_specs=[
            pl.BlockSpec((ROWS, D), lambda b: (b, b * 0)),
            pl.BlockSpec((ROWS, 1), lambda b: (b, b * 0)),
            pl.BlockSpec((1, D), lambda b: (b * 0, b * 0)),
            pl.BlockSpec((1, D), lambda b: (b * 0, b * 0)),
            pl.BlockSpec((D, BINS), lambda b: (b * 0, b * 0)),
        ],
        out_specs=[
            pl.BlockSpec((ROWS, D), lambda b: (b, b * 0)),
            pl.BlockSpec((ROWS, 1), lambda b: (b, b * 0)),
            pl.BlockSpec((ROWS, 1), lambda b: (b, b * 0)),
        ],
        out_shape=[
            jax.ShapeDtypeStruct((BATCH, D), jnp.bfloat16),
            jax.ShapeDtypeStruct((BATCH, 1), jnp.int32),
            jax.ShapeDtypeStruct((BATCH, 1), jnp.int32),
        ],
    )(features, env_col, new_mean, inv_std,
      random_projection.astype(jnp.float32))

    mesh = plsc.VectorSubcoreMesh(core_axis_name="c", subcore_axis_name="s")
    idx_tbl = jnp.arange(BATCH, dtype=jnp.int32).reshape(BATCH // 128, 128)
    sc = functools.partial(
        pl.kernel, mesh=mesh,
        compiler_params=pltpu.CompilerParams(needs_layout_passes=False),
        out_type=[
            jax.ShapeDtypeStruct((2, BATCH), jnp.float32),
            jax.ShapeDtypeStruct((NW, L), jnp.int32),
        ],
        scratch_types=[
            pltpu.VMEM((BATCH,), jnp.int32),
            pltpu.VMEM((BATCH,), jnp.int32),
            pltpu.VMEM((BATCH,), jnp.float32),
            pltpu.VMEM((BATCH // 128, 128), jnp.int32),
            pltpu.VMEM((KWAY * BPW,), jnp.int32),
            pltpu.VMEM((KWAY * BPW,), jnp.int32),
            pltpu.VMEM((L,), jnp.int32),
            pltpu.VMEM_SHARED((BATCH,), jnp.float32),
        ],
    )(_sc_body)
    parts, flags = sc(bkt.reshape(BATCH), tag.reshape(BATCH), idx_tbl)
    sc_rew = (parts[0] + parts[1]).reshape(BATCH, 1)
    rewards = lax.cond(jnp.any(flags != 0),
                       lambda: _tc_count(enc),
                       lambda: sc_rew)
    return rewards


# speedup vs baseline: 1.0574x; 1.0574x over previous
import functools

import jax
import jax.numpy as jnp
from jax import lax
from jax.experimental import pallas as pl
from jax.experimental.pallas import tpu as pltpu
from jax.experimental.pallas import tpu_sc as plsc

BATCH = 16384
D = 128
BINS = 32
NENV = 64
ROWS = 1024
NBLK = BATCH // ROWS

NW = 32
NBUCKET = 131072
BPW = NBUCKET // NW
KWAY = 4
L = 16
NCHUNK = BATCH // L


def _stats_kernel(f_ref, s1_ref, s2_ref):
    b = pl.program_id(0)
    x = f_ref[...]
    s1 = jnp.sum(x, axis=0)[None, :]
    s2 = jnp.sum(x * x, axis=0)[None, :]

    @pl.when(b == 0)
    def _():
        s1_ref[...] = s1
        s2_ref[...] = s2

    @pl.when(b > 0)
    def _():
        s1_ref[...] += s1
        s2_ref[...] += s2


def _hash_kernel(f_ref, env_ref, mean_ref, inv_ref, rp_ref, e_ref, b_ref,
                 t_ref):
    x = (f_ref[...] - mean_ref[...]) * inv_ref[...]
    p = jnp.dot(x, rp_ref[...], preferred_element_type=jnp.float32)
    sign = jnp.where(p > 0, jnp.float32(1), jnp.float32(-1))
    ks = lax.broadcasted_iota(jnp.int32, (1, NENV), 1)
    env = env_ref[...]
    onehot = jnp.where(env == ks, jnp.float32(8), jnp.float32(0))
    pad = jnp.zeros((ROWS, D - BINS - NENV), jnp.float32)
    e = jnp.concatenate([sign, onehot, pad], axis=1)
    e_ref[...] = e.astype(jnp.bfloat16)

    powers = jnp.left_shift(
        jnp.int32(1), lax.broadcasted_iota(jnp.int32, (1, BINS), 1))
    bits = jnp.where(p > 0, powers, jnp.int32(0))
    h = jnp.sum(bits, axis=1, keepdims=True, dtype=jnp.int32)
    b_ref[...] = jnp.bitwise_and(h, jnp.int32(NBUCKET - 1))
    hi = jnp.right_shift(h.astype(jnp.uint32), jnp.uint32(17)).astype(jnp.int32)
    t_ref[...] = jnp.bitwise_or(hi, jnp.left_shift(env, jnp.int32(15)))


def _count_kernel(ei_ref, ej_ref, out_ref):
    i = pl.program_id(0)
    j = pl.program_id(1)
    nj = pl.num_programs(1)

    @pl.when(j == 0)
    def _():
        out_ref[...] = jnp.ones((ROWS, 1), jnp.float32)

    @pl.when(j < i)
    def _():
        s = lax.dot_general(
            ei_ref[...], ej_ref[...], (((1,), (1,)), ((), ())),
            preferred_element_type=jnp.float32)
        out_ref[...] += jnp.sum((s > 95.0).astype(jnp.float32), axis=1,
                                keepdims=True)

    @pl.when(j == i)
    def _():
        s = lax.dot_general(
            ei_ref[...], ej_ref[...], (((1,), (1,)), ((), ())),
            preferred_element_type=jnp.float32)
        ii = lax.broadcasted_iota(jnp.int32, (ROWS, 1), 0)
        jj = lax.broadcasted_iota(jnp.int32, (1, ROWS), 1)
        eq = (s > 95.0) & (jj < ii)
        out_ref[...] += jnp.sum(eq.astype(jnp.float32), axis=1, keepdims=True)

    @pl.when(j == nj - 1)
    def _():
        out_ref[...] = 1.0 / jnp.sqrt(out_ref[...])


def _tc_count(enc):
    return pl.pallas_call(
        _count_kernel,
        grid=(NBLK, NBLK),
        in_specs=[
            pl.BlockSpec((ROWS, D), lambda i, j: (i, j * 0)),
            pl.BlockSpec((ROWS, D), lambda i, j: (j, i * 0)),
        ],
        out_specs=pl.BlockSpec((ROWS, 1), lambda i, j: (i, j * 0)),
        out_shape=jax.ShapeDtypeStruct((BATCH, 1), jnp.float32),
    )(enc, enc)


def _rsqrt_newton(c):
    ih = plsc.bitcast(c, jnp.int32)
    y = plsc.bitcast(jnp.int32(0x5F3759DF) - jnp.right_shift(ih, jnp.int32(1)),
                     jnp.float32)
    for _ in range(3):
        y = y * (1.5 - 0.5 * c * y * y)
    return y


def _sc_body(b_hbm, t_hbm, idx_hbm, out_hbm, flag_hbm,
             bkt_v, tag_v, rew_v, idx_v, ctab, ttab, stage_v, acc_sh, sem):
    info = plsc.get_sparse_core_info()
    nc = info.num_cores
    core = lax.axis_index("c")
    sid = lax.axis_index("s")
    wid = sid * nc + core

    pltpu.sync_copy(b_hbm, bkt_v)
    pltpu.sync_copy(t_hbm, tag_v)
    pltpu.sync_copy(idx_hbm, idx_v)

    zeros16i = jnp.zeros((L,), jnp.int32)

    def zero_body(z, carry):
        ctab[pl.ds(z * L, L)] = zeros16i
        return carry

    lax.fori_loop(jnp.int32(0), jnp.int32(KWAY * BPW // L), zero_body,
                  jnp.int32(0))

    iota16 = lax.broadcasted_iota(jnp.int32, (L,), 0)

    def chunk_body(c, flag):
        base = c * L
        b = bkt_v[pl.ds(base, L)]
        t = tag_v[pl.ds(base, L)]
        lb = jnp.bitwise_and(b, jnp.int32(BPW - 1))
        mine = jnp.right_shift(b, jnp.int32(12)) == wid

        gcs = []
        gts = []
        for k in range(KWAY):
            gidx = lb + jnp.int32(k * BPW)
            gcs.append(plsc.load_gather(ctab, [gidx]))
            gts.append(plsc.load_gather(ttab, [gidx]))
        occ = [g > 0 for g in gcs]
        match = [occ[k] & (gts[k] == t) for k in range(KWAY)]
        gathered = zeros16i
        found = iota16 < 0
        way = jnp.full((L,), KWAY, jnp.int32)
        for k in range(KWAY - 1, -1, -1):
            gathered = jnp.where(match[k], gcs[k], gathered)
            found = found | match[k]
            way = jnp.where(match[k], jnp.int32(k), way)

        pcv = plsc.all_reduce_population_count(mine)
        pc = jnp.max(pcv)

        def pairwise():
            rank = zeros16i
            after = zeros16i
            dbefore = zeros16i
            for m in range(L):
                sel = jnp.full((L,), m, jnp.int32) + base
                bm = plsc.load_gather(bkt_v, [sel])
                tm = plsc.load_gather(tag_v, [sel])
                beq = b == bm
                eq = beq & (t == tm)
                dif = beq & (t != tm)
                rank = rank + ((iota16 > m) & eq).astype(jnp.int32)
                after = after + ((iota16 < m) & eq).astype(jnp.int32)
                dbefore = dbefore + ((iota16 > m) & dif).astype(jnp.int32)
            return rank, after, dbefore

        def trivial():
            return zeros16i, zeros16i, zeros16i

        rank, after, dbefore = lax.cond(pc > 1, pairwise, trivial)

        is_last = after == 0
        cumempty = zeros16i
        claim = jnp.full((L,), KWAY, jnp.int32)
        for k in range(KWAY):
            cumempty = cumempty + (~occ[k]).astype(jnp.int32)
            take = (~occ[k]) & (cumempty == dbefore + 1) & (claim == KWAY)
            claim = jnp.where(take, jnp.int32(k), claim)
        slot = jnp.where(found, way, claim)
        overflow = mine & (slot == KWAY)
        flag = flag | overflow.astype(jnp.int32)

        newc = gathered + rank + 1
        upd = mine & is_last & (slot < KWAY)
        for k in range(KWAY):
            mk = upd & (slot == k)
            sidx = lb + jnp.int32(k * BPW)
            plsc.store_scatter(ctab, [sidx], newc, mask=mk)
            plsc.store_scatter(ttab, [sidx], t, mask=mk)

        rw = _rsqrt_newton(newc.astype(jnp.float32))
        rew_v[pl.ds(base, L)] = jnp.where(mine, rw, jnp.float32(0))
        return flag

    flag = lax.fori_loop(jnp.int32(0), jnp.int32(NCHUNK), chunk_body,
                         zeros16i)

    stage_v[...] = flag
    pltpu.sync_copy(stage_v, flag_hbm.at[wid])

    plsc.subcore_barrier()

    @pl.when(sid == 0)
    def _():
        handles = [
            pltpu.async_copy(rew_v.at[pl.ds(jnp.int32(j * 128), 128)],
                             acc_sh.at[idx_v.at[jnp.int32(j)]], sem)
            for j in range(BATCH // 128)
        ]
        for h in handles:
            h.wait()

    plsc.subcore_barrier()

    @pl.when(sid != 0)
    def _():
        handles = [
            pltpu.async_copy(rew_v.at[pl.ds(jnp.int32(j * 128), 128)],
                             acc_sh.at[idx_v.at[jnp.int32(j)]], sem, add=True)
            for j in range(BATCH // 128)
        ]
        for h in handles:
            h.wait()

    plsc.subcore_barrier()

    @pl.when(sid == 0)
    def _():
        pltpu.sync_copy(acc_sh, out_hbm.at[core])


def kernel(features, env_indices, random_projection):
    features = features.astype(jnp.float32)
    s1, s2 = pl.pallas_call(
        _stats_kernel,
        grid=(NBLK,),
        in_specs=[pl.BlockSpec((ROWS, D), lambda b: (b, b * 0))],
        out_specs=[
            pl.BlockSpec((1, D), lambda b: (b * 0, b * 0)),
            pl.BlockSpec((1, D), lambda b: (b * 0, b * 0)),
        ],
        out_shape=[
            jax.ShapeDtypeStruct((1, D), jnp.float32),
            jax.ShapeDtypeStruct((1, D), jnp.float32),
        ],
    )(features)

    eps_count = jnp.float32(1e-4)
    bc = jnp.float32(BATCH)
    batch_mean = s1 / bc
    batch_var = (s2 - s1 * s1 / bc) / (bc - 1.0)
    tot = eps_count + bc
    new_mean = batch_mean * (bc / tot)
    m2 = eps_count + batch_var * bc + batch_mean**2 * eps_count * bc / tot
    new_var = m2 / tot
    inv_std = 1.0 / jnp.sqrt(new_var + 1e-8)

    env_col = env_indices.astype(jnp.int32)[:, None]
    enc, bkt, tag = pl.pallas_call(
        _hash_kernel,
        grid=(NBLK,),
        in_specs=[
            pl.BlockSpec((ROWS, D), lambda b: (b, b * 0)),
            pl.BlockSpec((ROWS, 1), lambda b: (b, b * 0)),
            pl.BlockSpec((1, D), lambda b: (b * 0, b * 0)),
            pl.BlockSpec((1, D), lambda b: (b * 0, b * 0)),
            pl.BlockSpec((D, BINS), lambda b: (b * 0, b * 0)),
        ],
        out_specs=[
            pl.BlockSpec((ROWS, D), lambda b: (b, b * 0)),
            pl.BlockSpec((ROWS, 1), lambda b: (b, b * 0)),
            pl.BlockSpec((ROWS, 1), lambda b: (b, b * 0)),
        ],
        out_shape=[
            jax.ShapeDtypeStruct((BATCH, D), jnp.bfloat16),
            jax.ShapeDtypeStruct((BATCH, 1), jnp.int32),
            jax.ShapeDtypeStruct((BATCH, 1), jnp.int32),
        ],
    )(features, env_col, new_mean, inv_std,
      random_projection.astype(jnp.float32))

    mesh = plsc.VectorSubcoreMesh(core_axis_name="c", subcore_axis_name="s")
    idx_tbl = jnp.arange(BATCH, dtype=jnp.int32).reshape(BATCH // 128, 128)
    sc = functools.partial(
        pl.kernel, mesh=mesh,
        compiler_params=pltpu.CompilerParams(needs_layout_passes=False),
        out_type=[
            jax.ShapeDtypeStruct((2, BATCH), jnp.float32),
            jax.ShapeDtypeStruct((NW, L), jnp.int32),
        ],
        scratch_types=[
            pltpu.VMEM((BATCH,), jnp.int32),
            pltpu.VMEM((BATCH,), jnp.int32),
            pltpu.VMEM((BATCH,), jnp.float32),
            pltpu.VMEM((BATCH // 128, 128), jnp.int32),
            pltpu.VMEM((KWAY * BPW,), jnp.int32),
            pltpu.VMEM((KWAY * BPW,), jnp.int32),
            pltpu.VMEM((L,), jnp.int32),
            pltpu.VMEM_SHARED((BATCH,), jnp.float32),
            pltpu.SemaphoreType.DMA,
        ],
    )(_sc_body)
    parts, flags = sc(bkt.reshape(BATCH), tag.reshape(BATCH), idx_tbl)
    sc_rew = (parts[0] + parts[1]).reshape(BATCH, 1)
    rewards = lax.cond(jnp.any(flags != 0),
                       lambda: _tc_count(enc),
                       lambda: sc_rew)
    return rewards


# speedup vs baseline: 1.0606x; 1.0030x over previous
import functools

import jax
import jax.numpy as jnp
from jax import lax
from jax.experimental import pallas as pl
from jax.experimental.pallas import tpu as pltpu
from jax.experimental.pallas import tpu_sc as plsc

BATCH = 16384
D = 128
BINS = 32
NENV = 64
ROWS = 1024
NBLK = BATCH // ROWS

NW = 32
NBUCKET = 131072
BPW = NBUCKET // NW
KWAY = 4
L = 16
NCHUNK = BATCH // L


def _stats_kernel(f_ref, s1_ref, s2_ref):
    b = pl.program_id(0)
    x = f_ref[...]
    s1 = jnp.sum(x, axis=0)[None, :]
    s2 = jnp.sum(x * x, axis=0)[None, :]

    @pl.when(b == 0)
    def _():
        s1_ref[...] = s1
        s2_ref[...] = s2

    @pl.when(b > 0)
    def _():
        s1_ref[...] += s1
        s2_ref[...] += s2


def _hash_kernel(f_ref, env_ref, mean_ref, inv_ref, rp_ref, e_ref, b_ref,
                 t_ref):
    x = (f_ref[...] - mean_ref[...]) * inv_ref[...]
    p = jnp.dot(x, rp_ref[...], preferred_element_type=jnp.float32)
    sign = jnp.where(p > 0, jnp.float32(1), jnp.float32(-1))
    ks = lax.broadcasted_iota(jnp.int32, (1, NENV), 1)
    env = env_ref[...]
    onehot = jnp.where(env == ks, jnp.float32(8), jnp.float32(0))
    pad = jnp.zeros((ROWS, D - BINS - NENV), jnp.float32)
    e = jnp.concatenate([sign, onehot, pad], axis=1)
    e_ref[...] = e.astype(jnp.bfloat16)

    powers = jnp.left_shift(
        jnp.int32(1), lax.broadcasted_iota(jnp.int32, (1, BINS), 1))
    bits = jnp.where(p > 0, powers, jnp.int32(0))
    h = jnp.sum(bits, axis=1, keepdims=True, dtype=jnp.int32)
    b_ref[...] = jnp.bitwise_and(h, jnp.int32(NBUCKET - 1))
    hi = jnp.right_shift(h.astype(jnp.uint32), jnp.uint32(17)).astype(jnp.int32)
    t_ref[...] = jnp.bitwise_or(hi, jnp.left_shift(env, jnp.int32(15)))


def _count_kernel(ei_ref, ej_ref, out_ref):
    i = pl.program_id(0)
    j = pl.program_id(1)
    nj = pl.num_programs(1)

    @pl.when(j == 0)
    def _():
        out_ref[...] = jnp.ones((ROWS, 1), jnp.float32)

    @pl.when(j < i)
    def _():
        s = lax.dot_general(
            ei_ref[...], ej_ref[...], (((1,), (1,)), ((), ())),
            preferred_element_type=jnp.float32)
        out_ref[...] += jnp.sum((s > 95.0).astype(jnp.float32), axis=1,
                                keepdims=True)

    @pl.when(j == i)
    def _():
        s = lax.dot_general(
            ei_ref[...], ej_ref[...], (((1,), (1,)), ((), ())),
            preferred_element_type=jnp.float32)
        ii = lax.broadcasted_iota(jnp.int32, (ROWS, 1), 0)
        jj = lax.broadcasted_iota(jnp.int32, (1, ROWS), 1)
        eq = (s > 95.0) & (jj < ii)
        out_ref[...] += jnp.sum(eq.astype(jnp.float32), axis=1, keepdims=True)

    @pl.when(j == nj - 1)
    def _():
        out_ref[...] = 1.0 / jnp.sqrt(out_ref[...])


def _tc_count(enc):
    return pl.pallas_call(
        _count_kernel,
        grid=(NBLK, NBLK),
        in_specs=[
            pl.BlockSpec((ROWS, D), lambda i, j: (i, j * 0)),
            pl.BlockSpec((ROWS, D), lambda i, j: (j, i * 0)),
        ],
        out_specs=pl.BlockSpec((ROWS, 1), lambda i, j: (i, j * 0)),
        out_shape=jax.ShapeDtypeStruct((BATCH, 1), jnp.float32),
    )(enc, enc)


def _rsqrt_newton(c):
    ih = plsc.bitcast(c, jnp.int32)
    y = plsc.bitcast(jnp.int32(0x5F3759DF) - jnp.right_shift(ih, jnp.int32(1)),
                     jnp.float32)
    for _ in range(3):
        y = y * (1.5 - 0.5 * c * y * y)
    return y


def _sc_body(b_hbm, t_hbm, idx_hbm, out_hbm, flag_hbm,
             bkt_v, tag_v, rew_v, idx_v, ctab, ttab, stage_v, acc_sh, sem):
    info = plsc.get_sparse_core_info()
    nc = info.num_cores
    core = lax.axis_index("c")
    sid = lax.axis_index("s")
    wid = sid * nc + core

    pltpu.sync_copy(b_hbm, bkt_v)
    pltpu.sync_copy(t_hbm, tag_v)
    pltpu.sync_copy(idx_hbm, idx_v)

    zeros16i = jnp.zeros((L,), jnp.int32)

    def zero_body(z, carry):
        ctab[pl.ds(z * L, L)] = zeros16i
        return carry

    lax.fori_loop(jnp.int32(0), jnp.int32(KWAY * BPW // L), zero_body,
                  jnp.int32(0))

    iota16 = lax.broadcasted_iota(jnp.int32, (L,), 0)

    def chunk_body(c, flag):
        base = c * L
        b = bkt_v[pl.ds(base, L)]
        t = tag_v[pl.ds(base, L)]
        lb = jnp.bitwise_and(b, jnp.int32(BPW - 1))
        mine = jnp.right_shift(b, jnp.int32(12)) == wid

        pcv = plsc.all_reduce_population_count(mine)
        pc = jnp.max(pcv)

        def skip():
            rew_v[pl.ds(base, L)] = jnp.zeros((L,), jnp.float32)
            return flag

        def process():
            gcs = []
            gts = []
            for k in range(KWAY):
                gidx = lb + jnp.int32(k * BPW)
                gcs.append(plsc.load_gather(ctab, [gidx]))
                gts.append(plsc.load_gather(ttab, [gidx]))
            occ = [g > 0 for g in gcs]
            match = [occ[k] & (gts[k] == t) for k in range(KWAY)]
            gathered = zeros16i
            found = iota16 < 0
            way = jnp.full((L,), KWAY, jnp.int32)
            for k in range(KWAY - 1, -1, -1):
                gathered = jnp.where(match[k], gcs[k], gathered)
                found = found | match[k]
                way = jnp.where(match[k], jnp.int32(k), way)

            def pairwise():
                rank = zeros16i
                after = zeros16i
                dbefore = zeros16i
                for m in range(L):
                    sel = jnp.full((L,), m, jnp.int32) + base
                    bm = plsc.load_gather(bkt_v, [sel])
                    tm = plsc.load_gather(tag_v, [sel])
                    beq = b == bm
                    eq = beq & (t == tm)
                    dif = beq & (t != tm)
                    rank = rank + ((iota16 > m) & eq).astype(jnp.int32)
                    after = after + ((iota16 < m) & eq).astype(jnp.int32)
                    dbefore = dbefore + ((iota16 > m) & dif).astype(jnp.int32)
                return rank, after, dbefore

            def trivial():
                return zeros16i, zeros16i, zeros16i

            rank, after, dbefore = lax.cond(pc > 1, pairwise, trivial)

            is_last = after == 0
            cumempty = zeros16i
            claim = jnp.full((L,), KWAY, jnp.int32)
            for k in range(KWAY):
                cumempty = cumempty + (~occ[k]).astype(jnp.int32)
                take = (~occ[k]) & (cumempty == dbefore + 1) & (claim == KWAY)
                claim = jnp.where(take, jnp.int32(k), claim)
            slot = jnp.where(found, way, claim)
            overflow = mine & (slot == KWAY)
            nflag = flag | overflow.astype(jnp.int32)

            newc = gathered + rank + 1
            upd = mine & is_last & (slot < KWAY)
            for k in range(KWAY):
                mk = upd & (slot == k)
                sidx = lb + jnp.int32(k * BPW)
                plsc.store_scatter(ctab, [sidx], newc, mask=mk)
                plsc.store_scatter(ttab, [sidx], t, mask=mk)

            rw = _rsqrt_newton(newc.astype(jnp.float32))
            rew_v[pl.ds(base, L)] = jnp.where(mine, rw, jnp.float32(0))
            return nflag

        return lax.cond(pc == 0, skip, process)

    flag = lax.fori_loop(jnp.int32(0), jnp.int32(NCHUNK), chunk_body,
                         zeros16i)

    stage_v[...] = flag
    pltpu.sync_copy(stage_v, flag_hbm.at[wid])

    plsc.subcore_barrier()

    @pl.when(sid == 0)
    def _():
        handles = [
            pltpu.async_copy(rew_v.at[pl.ds(jnp.int32(j * 128), 128)],
                             acc_sh.at[idx_v.at[jnp.int32(j)]], sem)
            for j in range(BATCH // 128)
        ]
        for h in handles:
            h.wait()

    plsc.subcore_barrier()

    @pl.when(sid != 0)
    def _():
        handles = [
            pltpu.async_copy(rew_v.at[pl.ds(jnp.int32(j * 128), 128)],
                             acc_sh.at[idx_v.at[jnp.int32(j)]], sem, add=True)
            for j in range(BATCH // 128)
        ]
        for h in handles:
            h.wait()

    plsc.subcore_barrier()

    @pl.when(sid == 0)
    def _():
        pltpu.sync_copy(acc_sh, out_hbm.at[core])


def kernel(features, env_indices, random_projection):
    features = features.astype(jnp.float32)
    s1, s2 = pl.pallas_call(
        _stats_kernel,
        grid=(NBLK,),
        in_specs=[pl.BlockSpec((ROWS, D), lambda b: (b, b * 0))],
        out_specs=[
            pl.BlockSpec((1, D), lambda b: (b * 0, b * 0)),
            pl.BlockSpec((1, D), lambda b: (b * 0, b * 0)),
        ],
        out_shape=[
            jax.ShapeDtypeStruct((1, D), jnp.float32),
            jax.ShapeDtypeStruct((1, D), jnp.float32),
        ],
    )(features)

    eps_count = jnp.float32(1e-4)
    bc = jnp.float32(BATCH)
    batch_mean = s1 / bc
    batch_var = (s2 - s1 * s1 / bc) / (bc - 1.0)
    tot = eps_count + bc
    new_mean = batch_mean * (bc / tot)
    m2 = eps_count + batch_var * bc + batch_mean**2 * eps_count * bc / tot
    new_var = m2 / tot
    inv_std = 1.0 / jnp.sqrt(new_var + 1e-8)

    env_col = env_indices.astype(jnp.int32)[:, None]
    enc, bkt, tag = pl.pallas_call(
        _hash_kernel,
        grid=(NBLK,),
        in_specs=[
            pl.BlockSpec((ROWS, D), lambda b: (b, b * 0)),
            pl.BlockSpec((ROWS, 1), lambda b: (b, b * 0)),
            pl.BlockSpec((1, D), lambda b: (b * 0, b * 0)),
            pl.BlockSpec((1, D), lambda b: (b * 0, b * 0)),
            pl.BlockSpec((D, BINS), lambda b: (b * 0, b * 0)),
        ],
        out_specs=[
            pl.BlockSpec((ROWS, D), lambda b: (b, b * 0)),
            pl.BlockSpec((ROWS, 1), lambda b: (b, b * 0)),
            pl.BlockSpec((ROWS, 1), lambda b: (b, b * 0)),
        ],
        out_shape=[
            jax.ShapeDtypeStruct((BATCH, D), jnp.bfloat16),
            jax.ShapeDtypeStruct((BATCH, 1), jnp.int32),
            jax.ShapeDtypeStruct((BATCH, 1), jnp.int32),
        ],
    )(features, env_col, new_mean, inv_std,
      random_projection.astype(jnp.float32))

    mesh = plsc.VectorSubcoreMesh(core_axis_name="c", subcore_axis_name="s")
    idx_tbl = jnp.arange(BATCH, dtype=jnp.int32).reshape(BATCH // 128, 128)
    sc = functools.partial(
        pl.kernel, mesh=mesh,
        compiler_params=pltpu.CompilerParams(needs_layout_passes=False),
        out_type=[
            jax.ShapeDtypeStruct((2, BATCH), jnp.float32),
            jax.ShapeDtypeStruct((NW, L), jnp.int32),
        ],
        scratch_types=[
            pltpu.VMEM((BATCH,), jnp.int32),
            pltpu.VMEM((BATCH,), jnp.int32),
            pltpu.VMEM((BATCH,), jnp.float32),
            pltpu.VMEM((BATCH // 128, 128), jnp.int32),
            pltpu.VMEM((KWAY * BPW,), jnp.int32),
            pltpu.VMEM((KWAY * BPW,), jnp.int32),
            pltpu.VMEM((L,), jnp.int32),
            pltpu.VMEM_SHARED((BATCH,), jnp.float32),
            pltpu.SemaphoreType.DMA,
        ],
    )(_sc_body)
    parts, flags = sc(bkt.reshape(BATCH), tag.reshape(BATCH), idx_tbl)
    sc_rew = (parts[0] + parts[1]).reshape(BATCH, 1)
    rewards = lax.cond(jnp.any(flags != 0),
                       lambda: _tc_count(enc),
                       lambda: sc_rew)
    return rewards
